# baseline (speedup 1.0000x reference)
import functools

import jax
import jax.numpy as jnp
from jax import lax
from jax.experimental import pallas as pl
from jax.experimental.pallas import tpu as pltpu
from jax.experimental.pallas import tpu_sc as plsc

_B, _S, _D, _L = 4, 8192, 128, 16
_C = 128
_DEPTH = 4


@functools.cache
def _make_kernel():
    info = plsc.get_sparse_core_info()
    nc, ns = info.num_cores, info.num_subcores
    nw = nc * ns
    p_per_w = _S // nw
    n_items = _B * p_per_w // _C
    chunks_per_b = p_per_w // _C
    mesh = plsc.VectorSubcoreMesh(core_axis_name="c", subcore_axis_name="s")

    @functools.partial(
        pl.kernel,
        mesh=mesh,
        out_type=jax.ShapeDtypeStruct((_B, _S, _D), jnp.float32),
        scratch_types=[
            pltpu.VMEM((_B, p_per_w), jnp.int32),
            pltpu.VMEM((p_per_w, _D), jnp.float32),
            *[pltpu.VMEM((_C, _D), jnp.float32) for _ in range(_DEPTH)],
            pltpu.SemaphoreType.DMA,
            pltpu.SemaphoreType.DMA,
            *[pltpu.SemaphoreType.DMA for _ in range(_DEPTH)],
            *[pltpu.SemaphoreType.DMA for _ in range(_DEPTH)],
        ],
    )
    def k(idx_hbm, wemb_hbm, pemb_hbm, out_hbm,
          idx_v, pos_v, r0, r1, r2, r3, isem, psem,
          gs0, gs1, gs2, gs3, os0, os1, os2, os3):
        wid = lax.axis_index("s") * nc + lax.axis_index("c")
        pbase = wid * p_per_w
        rows = (r0, r1, r2, r3)
        gsem = (gs0, gs1, gs2, gs3)
        osem = (os0, os1, os2, os3)

        def item_idx(j):
            b, h = divmod(j, chunks_per_b)
            return b, h

        def gather(j):
            b, h = item_idx(j)
            return pltpu.async_copy(
                wemb_hbm.at[idx_v.at[b, pl.ds(h * _C, _C)]],
                rows[j % _DEPTH], gsem[j % _DEPTH])

        icopy = pltpu.async_copy(
            idx_hbm.at[:, pl.ds(pbase, p_per_w)], idx_v, isem)
        pcopy = pltpu.async_copy(
            pemb_hbm.at[pl.ds(pbase, p_per_w)], pos_v, psem)
        with jax.named_scope("iwait"):
            icopy.wait()
        with jax.named_scope("prime"):
            gcur = [gather(j) for j in range(_DEPTH)]
        with jax.named_scope("pwait"):
            pcopy.wait()

        ocur = [None] * _DEPTH
        for j in range(n_items):
            buf = j % _DEPTH
            b, h = item_idx(j)
            with jax.named_scope(f"gwait{j}"):
                gcur[buf].wait()
            rbuf = rows[buf]
            prow = h * _C

            with jax.named_scope(f"add{j}"):
                @plsc.parallel_loop(0, _C, unroll=4)
                def add_body(r, rbuf=rbuf, prow=prow):
                    for c in range(_D // _L):
                        sl = pl.ds(c * _L, _L)
                        plsc.addupdate(rbuf.at[r, sl], pos_v[prow + r, sl])

            ocur[buf] = pltpu.async_copy(
                rbuf, out_hbm.at[b, pl.ds(pbase + prow, _C)], osem[buf])
            nxt = j + 2
            if _DEPTH <= nxt < n_items:
                with jax.named_scope(f"owait{j}"):
                    ocur[nxt % _DEPTH].wait()
                gcur[nxt % _DEPTH] = gather(nxt)
        with jax.named_scope("drain"):
            for buf in range(_DEPTH):
                ocur[buf].wait()

    return k


def kernel(input_ids, word_embeddings, position_embeddings):
    if input_ids.dtype != jnp.int32:
        input_ids = input_ids.astype(jnp.int32)
    return _make_kernel()(input_ids, word_embeddings, position_embeddings)


# speedup vs baseline: 1.0026x; 1.0026x over previous
import functools

import jax
import jax.numpy as jnp
from jax import lax
from jax.experimental import pallas as pl
from jax.experimental.pallas import tpu as pltpu
from jax.experimental.pallas import tpu_sc as plsc

_B, _S, _D, _L = 4, 8192, 128, 16
_C = 128
_DEPTH = 4


@functools.cache
def _make_kernel():
    info = plsc.get_sparse_core_info()
    nc, ns = info.num_cores, info.num_subcores
    nw = nc * ns
    p_per_w = _S // nw
    n_items = _B * p_per_w // _C
    n_halves = p_per_w // _C
    mesh = plsc.VectorSubcoreMesh(core_axis_name="c", subcore_axis_name="s")

    @functools.partial(
        pl.kernel,
        mesh=mesh,
        out_type=jax.ShapeDtypeStruct((_B, _S, _D), jnp.float32),
        scratch_types=[
            pltpu.VMEM((_B, p_per_w), jnp.int32),
            pltpu.VMEM((p_per_w, _D), jnp.float32),
            *[pltpu.VMEM((_C, _D), jnp.float32) for _ in range(_DEPTH)],
            pltpu.SemaphoreType.DMA,
            *[pltpu.SemaphoreType.DMA for _ in range(n_halves)],
            *[pltpu.SemaphoreType.DMA for _ in range(_DEPTH)],
            *[pltpu.SemaphoreType.DMA for _ in range(_DEPTH)],
        ],
    )
    def k(idx_hbm, wemb_hbm, pemb_hbm, out_hbm,
          idx_v, pos_v, r0, r1, r2, r3, isem, ps0, ps1,
          gs0, gs1, gs2, gs3, os0, os1, os2, os3):
        wid = lax.axis_index("s") * nc + lax.axis_index("c")
        pbase = wid * p_per_w
        rows = (r0, r1, r2, r3)
        psem = (ps0, ps1)
        gsem = (gs0, gs1, gs2, gs3)
        osem = (os0, os1, os2, os3)

        def item_idx(j):
            h, b = divmod(j, _B)
            return b, h

        def gather(j):
            b, h = item_idx(j)
            return pltpu.async_copy(
                wemb_hbm.at[idx_v.at[b, pl.ds(h * _C, _C)]],
                rows[j % _DEPTH], gsem[j % _DEPTH])

        pcopy = [pltpu.async_copy(
            pemb_hbm.at[pl.ds(pbase + h * _C, _C)],
            pos_v.at[pl.ds(h * _C, _C)], psem[h]) for h in range(n_halves)]
        icopy = pltpu.async_copy(
            idx_hbm.at[:, pl.ds(pbase, p_per_w)], idx_v, isem)
        icopy.wait()
        gcur = [gather(j) for j in range(_DEPTH)]

        ocur = [None] * _DEPTH
        for j in range(n_items):
            buf = j % _DEPTH
            b, h = item_idx(j)
            if j % _B == 0:
                pcopy[h].wait()
            gcur[buf].wait()
            rbuf = rows[buf]
            prow = h * _C

            @plsc.parallel_loop(0, _C, unroll=4)
            def add_body(r, rbuf=rbuf, prow=prow):
                for c in range(_D // _L):
                    sl = pl.ds(c * _L, _L)
                    plsc.addupdate(rbuf.at[r, sl], pos_v[prow + r, sl])

            ocur[buf] = pltpu.async_copy(
                rbuf, out_hbm.at[b, pl.ds(pbase + prow, _C)], osem[buf])
            nxt = j + 2
            if _DEPTH <= nxt < n_items:
                ocur[nxt % _DEPTH].wait()
                gcur[nxt % _DEPTH] = gather(nxt)
        for buf in range(_DEPTH):
            ocur[buf].wait()

    return k


def kernel(input_ids, word_embeddings, position_embeddings):
    if input_ids.dtype != jnp.int32:
        input_ids = input_ids.astype(jnp.int32)
    return _make_kernel()(input_ids, word_embeddings, position_embeddings)


# speedup vs baseline: 1.0544x; 1.0517x over previous
import functools

import jax
import jax.numpy as jnp
from jax import lax
from jax.experimental import pallas as pl
from jax.experimental.pallas import tpu as pltpu
from jax.experimental.pallas import tpu_sc as plsc

_B, _S, _D, _L = 4, 8192, 128, 16
_C = 128
_DEPTH = 4


@functools.cache
def _make_kernel():
    info = plsc.get_sparse_core_info()
    nc, ns = info.num_cores, info.num_subcores
    nw = nc * ns
    p_per_w = _S // nw
    n_items = _B * p_per_w // _C
    n_halves = p_per_w // _C
    mesh = plsc.VectorSubcoreMesh(core_axis_name="c", subcore_axis_name="s")

    @functools.partial(
        pl.kernel,
        mesh=mesh,
        out_type=jax.ShapeDtypeStruct((_B, _S, _D), jnp.float32),
        scratch_types=[
            pltpu.VMEM((_B, p_per_w), jnp.int32),
            pltpu.VMEM((p_per_w, _D), jnp.float32),
            pltpu.VMEM((_DEPTH, _C, _D), jnp.float32),
            pltpu.SemaphoreType.DMA,
            pltpu.SemaphoreType.DMA((n_halves,)),
            pltpu.SemaphoreType.DMA((_DEPTH,)),
            pltpu.SemaphoreType.DMA((_DEPTH,)),
        ],
    )
    def k(idx_hbm, wemb_hbm, pemb_hbm, out_hbm,
          idx_v, pos_v, rows_v, isem, psem, gsem, osem):
        wid = lax.axis_index("s") * nc + lax.axis_index("c")
        pbase = wid * p_per_w

        def coords(j):
            return lax.rem(j, _B), j // _B

        def pos_desc(h):
            return pltpu.make_async_copy(
                pemb_hbm.at[pl.ds(pbase + h * _C, _C)],
                pos_v.at[pl.ds(h * _C, _C)], psem.at[h])

        def gather_desc(j):
            b, h = coords(j)
            buf = lax.rem(j, _DEPTH)
            return pltpu.make_async_copy(
                wemb_hbm.at[idx_v.at[b, pl.ds(h * _C, _C)]],
                rows_v.at[buf], gsem.at[buf])

        def out_desc(j):
            b, h = coords(j)
            buf = lax.rem(j, _DEPTH)
            return pltpu.make_async_copy(
                rows_v.at[buf], out_hbm.at[b, pl.ds(pbase + h * _C, _C)],
                osem.at[buf])

        for h in range(n_halves):
            pos_desc(h).start()
        pltpu.sync_copy(idx_hbm.at[:, pl.ds(pbase, p_per_w)], idx_v)

        def prime(j, c):
            gather_desc(j).start()
            return c
        lax.fori_loop(0, _DEPTH, prime, 0)

        def item(j, c):
            b, h = coords(j)
            buf = lax.rem(j, _DEPTH)

            @pl.when(lax.rem(j, _B) == 0)
            def _():
                pos_desc(h).wait()

            gather_desc(j).wait()

            @plsc.parallel_loop(0, _C, unroll=4)
            def add_body(r):
                prow = h * _C + r
                for col in range(_D // _L):
                    sl = pl.ds(col * _L, _L)
                    plsc.addupdate(rows_v.at[buf, r, sl], pos_v[prow, sl])

            out_desc(j).start()
            nxt = j + 2

            @pl.when(jnp.logical_and(nxt >= _DEPTH, nxt < n_items))
            def _():
                out_desc(nxt - _DEPTH).wait()
                gather_desc(nxt).start()

            return c
        lax.fori_loop(0, n_items, item, 0)

        def drain(j, c):
            out_desc(j).wait()
            return c
        lax.fori_loop(n_items - _DEPTH, n_items, drain, 0)

    return k


def kernel(input_ids, word_embeddings, position_embeddings):
    if input_ids.dtype != jnp.int32:
        input_ids = input_ids.astype(jnp.int32)
    return _make_kernel()(input_ids, word_embeddings, position_embeddings)
